# baseline (speedup 1.0000x reference)
import functools
import math

import jax
import jax.numpy as jnp
from jax import lax
from jax.experimental import pallas as pl
from jax.experimental.pallas import tpu as pltpu
from jax.experimental.pallas import tpu_sc as plsc

_NUM_CORES = 2
_NUM_SUBCORES = 16
_NUM_WORKERS = _NUM_CORES * _NUM_SUBCORES

_CHUNK = 128
_NBUF = 4


def _pack_table(table):
    v, d = table.shape
    z = table.reshape(v, d // 32, 2, 16).swapaxes(-2, -1).astype(jnp.bfloat16)
    return lax.bitcast_convert_type(z, jnp.int32).reshape(v, d // 2)


@functools.cache
def _make_gather(v, d, n):
    per_w = n // _NUM_WORKERS
    nchunks = per_w // _CHUNK
    assert nchunks % _NBUF == 0
    nw = d // 32
    mesh = plsc.VectorSubcoreMesh(
        core_axis_name="c", subcore_axis_name="s", num_cores=_NUM_CORES
    )

    scale = jnp.float32(math.sqrt(d))

    @functools.partial(
        pl.kernel,
        mesh=mesh,
        out_type=jax.ShapeDtypeStruct((n, d), jnp.float32),
        compiler_params=pltpu.CompilerParams(needs_layout_passes=False, use_tc_tiling_on_sc=False),
        scratch_types=[
            pltpu.VMEM((nchunks, _CHUNK), jnp.int32),
            pltpu.VMEM((_NBUF, _CHUNK, d // 2), jnp.int32),
            pltpu.VMEM((_NBUF, _CHUNK, d), jnp.float32),
            pltpu.SemaphoreType.DMA,
            pltpu.SemaphoreType.DMA((_NBUF,)),
            pltpu.SemaphoreType.DMA((_NBUF,)),
        ],
    )
    def gather_kernel(
        table_hbm, idx_hbm, out_hbm, idx_v, in_v, out_v, isem, gsem, ssem
    ):
        wid = lax.axis_index("s") * _NUM_CORES + lax.axis_index("c")
        base_chunk = wid * nchunks
        base_row = wid * per_w

        pltpu.async_copy(idx_hbm.at[pl.ds(base_chunk, nchunks)], idx_v, isem).wait()

        for b in range(_NBUF):
            pltpu.async_copy(table_hbm.at[idx_v.at[b]], in_v.at[b], gsem.at[b])

        @pl.loop(0, nchunks, step=_NBUF)
        def _round(j0):
            for b in range(_NBUF):
                j = j0 + b
                pltpu.make_async_copy(
                    table_hbm.at[idx_v.at[b]], in_v.at[b], gsem.at[b]
                ).wait()
                @pl.when(j >= _NBUF)
                def _drain_out():
                    pltpu.make_async_copy(
                        out_v.at[b],
                        out_hbm.at[pl.ds(base_row + (j - _NBUF) * _CHUNK, _CHUNK)],
                        ssem.at[b],
                    ).wait()

                @pl.loop(0, _CHUNK, unroll=4)
                def _convert(r):
                    for c in range(nw):
                        w = in_v[b, r, pl.ds(c * 16, 16)]
                        x = plsc.bitcast(w, jnp.bfloat16)
                        lo, hi = plsc.unpack(x, format=plsc.PackFormat.INTERLEAVED)
                        out_v[b, r, pl.ds(c * 32, 16)] = lo * scale
                        out_v[b, r, pl.ds(c * 32 + 16, 16)] = hi * scale

                pltpu.make_async_copy(
                    out_v.at[b],
                    out_hbm.at[pl.ds(base_row + j * _CHUNK, _CHUNK)],
                    ssem.at[b],
                ).start()
                jn = j + _NBUF

                @pl.when(jn < nchunks)
                def _refill():
                    pltpu.async_copy(
                        table_hbm.at[idx_v.at[jn]], in_v.at[b], gsem.at[b]
                    )

        for b in range(_NBUF):
            pltpu.make_async_copy(
                out_v.at[b],
                out_hbm.at[pl.ds(base_row + (nchunks - _NBUF + b) * _CHUNK, _CHUNK)],
                ssem.at[b],
            ).wait()

    return gather_kernel


def kernel(tok_ids, emb_table):
    b, l = tok_ids.shape
    v, d = emb_table.shape
    n = b * l
    packed = _pack_table(emb_table)
    idx2d = tok_ids.reshape(n // _CHUNK, _CHUNK)
    out = _make_gather(v, d, n)(packed, idx2d)
    return out.reshape(b, l, d)
